# baseline (speedup 1.0000x reference)
import jax
import jax.numpy as jnp
from jax.experimental import pallas as pl


def _sdhgcn_body(adj_ref, x_ref, w_ref, out_ref):
    a16 = adj_ref[...].astype(jnp.bfloat16)
    xw = jnp.dot(x_ref[...], w_ref[...],
                 preferred_element_type=jnp.float32)
    xw16 = xw.astype(jnp.bfloat16)
    out_t = jax.lax.dot_general(
        xw16, a16, dimension_numbers=(((0,), (0,)), ((), ())),
        preferred_element_type=jnp.float32)
    ones16 = jnp.ones((a16.shape[0], 8), jnp.bfloat16)
    deg = jax.lax.dot_general(
        ones16, a16, dimension_numbers=(((0,), (0,)), ((), ())),
        preferred_element_type=jnp.float32)[:1, :]
    coeff = jax.lax.rsqrt(jnp.maximum(deg, 1.0))
    out_ref[...] = jnp.maximum(out_t * coeff, 0.0).T


def kernel(X, adj_matrix, weight):
    n, d_out = X.shape[0], weight.shape[1]
    return pl.pallas_call(
        _sdhgcn_body,
        out_shape=jax.ShapeDtypeStruct((n, d_out), jnp.float32),
    )(adj_matrix, X, weight)
